# baseline (speedup 1.0000x reference)
import functools

import jax
import jax.numpy as jnp
from jax import lax
from jax.experimental import pallas as pl
from jax.experimental.pallas import tpu as pltpu
from jax.experimental.pallas import tpu_sc as plsc

DIM = 32
GRP = 128
TPAD = 131

_info = plsc.get_sparse_core_info()
_NC, _NS = _info.num_cores, _info.num_subcores
_NW = _NC * _NS


@jax.jit
def _sc_relayout(table_t):
    v_size = table_t.shape[1]
    n_full = v_size // GRP
    tail = v_size - n_full * GRP
    n_blk = n_full + 1
    n_steps = ((n_blk + 2 * _NW - 1) // (2 * _NW)) * 2
    mesh = plsc.VectorSubcoreMesh(core_axis_name="c", subcore_axis_name="s")

    @functools.partial(
        pl.kernel,
        out_type=jax.ShapeDtypeStruct((n_blk * DIM, GRP), jnp.float32),
        mesh=mesh,
        scratch_types=[
            pltpu.VMEM((DIM, GRP), jnp.float32),
            pltpu.VMEM((DIM, GRP), jnp.float32),
            pltpu.VMEM((DIM, GRP), jnp.float32),
            pltpu.VMEM((DIM, GRP), jnp.float32),
            pltpu.SemaphoreType.DMA,
            pltpu.SemaphoreType.DMA,
            pltpu.SemaphoreType.DMA,
            pltpu.SemaphoreType.DMA,
        ],
        compiler_params=pltpu.CompilerParams(
            use_tc_tiling_on_sc=True, needs_layout_passes=False,
            disable_bounds_checks=True),
    )
    def k(tt_hbm, out_hbm, b0, b1, s0, s1, gs0, gs1, ss0, ss1):
        blks, slabs = (b0, b1), (s0, s1)
        gsems, ssems = (gs0, gs1), (ss0, ss1)
        wid = lax.axis_index("s") * _NC + lax.axis_index("c")

        iota16 = lax.broadcasted_iota(jnp.int32, (16,), 0)
        rows_h = [(iota16 + 16 * h) // 4 for h in range(8)]
        cols_h = [((iota16 + 16 * h) % 4) * DIM for h in range(8)]

        def ceff(step):
            return jnp.minimum(step * _NW + wid, n_full)

        def load(c, p):
            pltpu.async_copy(
                tt_hbm.at[:, pl.ds(c * GRP, GRP)], blks[p], gsems[p])

        def load_wait(c, p):
            pltpu.make_async_copy(
                tt_hbm.at[:, pl.ds(0, GRP)], blks[p], gsems[p]).wait()

        def transpose(p):
            bv, sv = blks[p], slabs[p]
            for d in range(DIM):
                for h in range(8):
                    v = bv[d, pl.ds(16 * h, 16)]
                    plsc.store_scatter(sv, [rows_h[h], cols_h[h] + d], v)

        def store(c, p):
            pltpu.async_copy(
                slabs[p], out_hbm.at[pl.ds(c * DIM, DIM), :], ssems[p])

        def store_wait(p):
            pltpu.make_async_copy(
                slabs[p], out_hbm.at[pl.ds(0, DIM), :], ssems[p]).wait()

        load(ceff(0), 0)
        load(ceff(1), 1)

        def body(i, carry):
            for p in range(2):
                step = 2 * i + p
                c = ceff(step)
                load_wait(c, p)

                @pl.when(i >= 1)
                def _():
                    store_wait(p)

                transpose(p)

                @pl.when(i < n_steps // 2 - 1)
                def _():
                    load(ceff(step + 2), p)

                store(c, p)
            return carry

        lax.fori_loop(0, n_steps // 2, body, 0)
        store_wait(0)
        store_wait(1)

    return k(table_t)


@functools.partial(jax.jit, static_argnums=(2, 3))
def _sc_gather(ids_lin, table_lin, seq, nb):
    n_groups = seq * nb
    per_w = n_groups // _NW
    mesh = plsc.VectorSubcoreMesh(core_axis_name="c", subcore_axis_name="s")

    @functools.partial(
        pl.kernel,
        out_type=jax.ShapeDtypeStruct((seq, DIM // 8, nb, 8, GRP), jnp.float32),
        mesh=mesh,
        scratch_types=[
            pltpu.VMEM((per_w * GRP,), jnp.int32),
            pltpu.VMEM((GRP, DIM), jnp.float32),
            pltpu.VMEM((GRP, DIM), jnp.float32),
            pltpu.VMEM((DIM, TPAD), jnp.float32),
            pltpu.VMEM((DIM, TPAD), jnp.float32),
            pltpu.SemaphoreType.DMA,
            pltpu.SemaphoreType.DMA,
            pltpu.SemaphoreType.DMA,
            pltpu.SemaphoreType.DMA,
        ],
        compiler_params=pltpu.CompilerParams(
            use_tc_tiling_on_sc=False, needs_layout_passes=False),
    )
    def k(ids_hbm, tab_hbm, out_hbm, idx_v, r0, r1, t0, t1, gs0, gs1, ss0, ss1):
        rows, tiles = (r0, r1), (t0, t1)
        gsems, ssems = (gs0, gs1), (ss0, ss1)
        wid = lax.axis_index("s") * _NC + lax.axis_index("c")
        gbase = wid * per_w
        pltpu.sync_copy(ids_hbm.at[pl.ds(gbase * GRP, per_w * GRP)], idx_v)

        iota16 = lax.broadcasted_iota(jnp.int32, (16,), 0)
        dvec = [iota16 + 16 * h for h in range(2)]
        zero16 = jnp.zeros((16,), jnp.int32)

        def gather(g, p):
            pltpu.async_copy(
                tab_hbm.at[idx_v.at[pl.ds(g * GRP, GRP)]], rows[p], gsems[p])

        def gather_wait(p):
            pltpu.make_async_copy(
                tab_hbm.at[pl.ds(0, GRP)], rows[p], gsems[p]).wait()

        def transpose(p):
            rv, tv = rows[p], tiles[p]
            for b in range(GRP):
                bidx = zero16 + b
                for h in range(2):
                    v = rv[b, pl.ds(16 * h, 16)]
                    plsc.store_scatter(tv, [dvec[h], bidx], v)

        def store(g, p):
            s = (gbase + g) // nb
            b = (gbase + g) % nb
            for j in range(DIM // 8):
                pltpu.async_copy(
                    tiles[p].at[pl.ds(8 * j, 8), pl.ds(0, GRP)],
                    out_hbm.at[s, j, b], ssems[p])

        def store_wait(p):
            for j in range(DIM // 8):
                pltpu.make_async_copy(
                    tiles[p].at[pl.ds(8 * j, 8), pl.ds(0, GRP)],
                    out_hbm.at[0, j, 0], ssems[p]).wait()

        gather(0, 0)
        gather(1, 1)

        def body(i, carry):
            for p in range(2):
                g = 2 * i + p
                gather_wait(p)

                @pl.when(i >= 1)
                def _():
                    store_wait(p)

                transpose(p)

                @pl.when(g + 2 < per_w)
                def _():
                    gather(g + 2, p)

                store(g, p)
            return carry

        lax.fori_loop(0, per_w // 2, body, 0)
        store_wait(0)
        store_wait(1)

    return k(ids_lin, table_lin)


def kernel(input_ids, table):
    bsz, seq = input_ids.shape
    vocab = table.shape[0]
    nb = bsz // GRP
    ids_lin = input_ids.T.reshape(-1)
    lin_blocks = _sc_relayout(table.T)
    table_lin = lin_blocks.reshape(-1, DIM)
    arr = _sc_gather(ids_lin, table_lin, seq, nb)
    out = arr.transpose(2, 4, 0, 1, 3).reshape(bsz, seq, DIM)
    return out


# speedup vs baseline: 1.2143x; 1.2143x over previous
import functools

import jax
import jax.numpy as jnp
from jax import lax
from jax.experimental import pallas as pl
from jax.experimental.pallas import tpu as pltpu
from jax.experimental.pallas import tpu_sc as plsc

DIM = 32
GRP = 128
TPAD = 131

_info = plsc.get_sparse_core_info()
_NC, _NS = _info.num_cores, _info.num_subcores
_NW = _NC * _NS


@jax.jit
def _sc_relayout(table_t):
    v_size = table_t.shape[1]
    n_full = v_size // GRP
    tail = v_size - n_full * GRP
    n_blk = n_full + 1
    n_steps = ((n_blk + 2 * _NW - 1) // (2 * _NW)) * 2
    mesh = plsc.VectorSubcoreMesh(core_axis_name="c", subcore_axis_name="s")

    @functools.partial(
        pl.kernel,
        out_type=jax.ShapeDtypeStruct((n_blk * DIM, GRP), jnp.float32),
        mesh=mesh,
        scratch_types=[
            pltpu.VMEM((DIM, GRP), jnp.float32),
            pltpu.VMEM((DIM, GRP), jnp.float32),
            pltpu.VMEM((DIM, GRP), jnp.float32),
            pltpu.VMEM((DIM, GRP), jnp.float32),
            pltpu.SemaphoreType.DMA,
            pltpu.SemaphoreType.DMA,
            pltpu.SemaphoreType.DMA,
            pltpu.SemaphoreType.DMA,
        ],
        compiler_params=pltpu.CompilerParams(
            use_tc_tiling_on_sc=True, needs_layout_passes=False,
            disable_bounds_checks=True),
    )
    def k(tt_hbm, out_hbm, b0, b1, s0, s1, gs0, gs1, ss0, ss1):
        blks, slabs = (b0, b1), (s0, s1)
        gsems, ssems = (gs0, gs1), (ss0, ss1)
        wid = lax.axis_index("s") * _NC + lax.axis_index("c")

        iota16 = lax.broadcasted_iota(jnp.int32, (16,), 0)
        rows_h = [(iota16 + 16 * h) // 4 for h in range(8)]
        cols_h = [((iota16 + 16 * h) % 4) * DIM for h in range(8)]

        def ceff(step):
            return jnp.minimum(step * _NW + wid, n_full)

        def load(c, p):
            pltpu.async_copy(
                tt_hbm.at[:, pl.ds(c * GRP, GRP)], blks[p], gsems[p])

        def load_wait(c, p):
            pltpu.make_async_copy(
                tt_hbm.at[:, pl.ds(0, GRP)], blks[p], gsems[p]).wait()

        def transpose(p):
            bv, sv = blks[p], slabs[p]
            for d in range(DIM):
                for h in range(8):
                    v = bv[d, pl.ds(16 * h, 16)]
                    plsc.store_scatter(sv, [rows_h[h], cols_h[h] + d], v)

        def store(c, p):
            pltpu.async_copy(
                slabs[p], out_hbm.at[pl.ds(c * DIM, DIM), :], ssems[p])

        def store_wait(p):
            pltpu.make_async_copy(
                slabs[p], out_hbm.at[pl.ds(0, DIM), :], ssems[p]).wait()

        load(ceff(0), 0)
        load(ceff(1), 1)

        def body(i, carry):
            for p in range(2):
                step = 2 * i + p
                c = ceff(step)
                load_wait(c, p)

                @pl.when(i >= 1)
                def _():
                    store_wait(p)

                transpose(p)

                @pl.when(i < n_steps // 2 - 1)
                def _():
                    load(ceff(step + 2), p)

                store(c, p)
            return carry

        lax.fori_loop(0, n_steps // 2, body, 0)
        store_wait(0)
        store_wait(1)

    return k(table_t)


@functools.partial(jax.jit, static_argnums=(2, 3))
def _sc_gather(ids_lin, table_lin, seq, nb):
    n_groups = seq * nb
    per_w = n_groups // _NW
    mesh = plsc.VectorSubcoreMesh(core_axis_name="c", subcore_axis_name="s")

    @functools.partial(
        pl.kernel,
        out_type=jax.ShapeDtypeStruct((seq, DIM // 8, nb, 8, GRP), jnp.float32),
        mesh=mesh,
        scratch_types=[
            pltpu.VMEM((per_w * GRP,), jnp.int32),
            pltpu.VMEM((GRP, DIM), jnp.float32),
            pltpu.VMEM((GRP, DIM), jnp.float32),
            pltpu.VMEM((DIM, TPAD), jnp.float32),
            pltpu.VMEM((DIM, TPAD), jnp.float32),
            pltpu.SemaphoreType.DMA,
            pltpu.SemaphoreType.DMA,
            pltpu.SemaphoreType.DMA,
            pltpu.SemaphoreType.DMA,
        ],
        compiler_params=pltpu.CompilerParams(
            use_tc_tiling_on_sc=False, needs_layout_passes=False),
    )
    def k(ids_hbm, tab_hbm, out_hbm, idx_v, r0, r1, t0, t1, gs0, gs1, ss0, ss1):
        rows, tiles = (r0, r1), (t0, t1)
        gsems, ssems = (gs0, gs1), (ss0, ss1)
        wid = lax.axis_index("s") * _NC + lax.axis_index("c")
        gbase = wid * per_w
        pltpu.sync_copy(ids_hbm.at[pl.ds(gbase * GRP, per_w * GRP)], idx_v)

        iota16 = lax.broadcasted_iota(jnp.int32, (16,), 0)
        dvec = [iota16 + 16 * h for h in range(2)]
        zero16 = jnp.zeros((16,), jnp.int32)

        def gather(g, p):
            pltpu.async_copy(
                tab_hbm.at[idx_v.at[pl.ds(g * GRP, GRP)]], rows[p], gsems[p])

        def gather_wait(p):
            pltpu.make_async_copy(
                tab_hbm.at[pl.ds(0, GRP)], rows[p], gsems[p]).wait()

        def transpose(p):
            rv, tv = rows[p], tiles[p]
            for b in range(GRP):
                bidx = zero16 + b
                for h in range(2):
                    v = rv[b, pl.ds(16 * h, 16)]
                    plsc.store_scatter(tv, [dvec[h], bidx], v)

        def store(g, p):
            s = (gbase + g) // nb
            b = (gbase + g) % nb
            for j in range(DIM // 8):
                pltpu.async_copy(
                    tiles[p].at[pl.ds(8 * j, 8), pl.ds(0, GRP)],
                    out_hbm.at[s, j, b], ssems[p])

        def store_wait(p):
            for j in range(DIM // 8):
                pltpu.make_async_copy(
                    tiles[p].at[pl.ds(8 * j, 8), pl.ds(0, GRP)],
                    out_hbm.at[0, j, 0], ssems[p]).wait()

        gather(0, 0)
        gather(1, 1)

        def body(i, carry):
            for p in range(2):
                g = 2 * i + p
                gather_wait(p)

                @pl.when(i >= 1)
                def _():
                    store_wait(p)

                transpose(p)

                @pl.when(g + 2 < per_w)
                def _():
                    gather(g + 2, p)

                store(g, p)
            return carry

        lax.fori_loop(0, per_w // 2, body, 0)
        store_wait(0)
        store_wait(1)

    return k(ids_lin, table_lin)


def kernel(input_ids, table):
    bsz, seq = input_ids.shape
    vocab = table.shape[0]
    nb = bsz // GRP
    ids_lin = (input_ids * 4).T.reshape(-1)
    padded = jnp.pad(table, ((0, 0), (0, GRP - DIM)))
    table_lin = padded.reshape(-1, DIM)
    arr = _sc_gather(ids_lin, table_lin, seq, nb)
    out = arr.transpose(2, 4, 0, 1, 3).reshape(bsz, seq, DIM)
    return out
